# baseline (speedup 1.0000x reference)
import functools

import jax
import jax.numpy as jnp
from jax import lax
from jax.experimental import pallas as pl
from jax.experimental.pallas import tpu as pltpu
from jax.experimental.pallas import tpu_sc as plsc

N = 10000
D = 128
DH = D // 2
E = 320000
K = 2
NC, NS = 2, 16
B = 128
NBLK = 160
CB = 16
NCHK = NBLK // CB
EPAD = NS * NBLK * B
NP = 10240
ROWS_PER_SUB = NP // NS


def _linear_relu_split(xp, Wt_s, b_s):
    def body(x_ref, w_ref, b_ref, o_ref):
        acc = jnp.dot(x_ref[...], w_ref[0], preferred_element_type=jnp.float32)
        o_ref[0] = jnp.maximum(acc + b_ref[0], 0.0)

    return pl.pallas_call(
        body,
        grid=(2, 2),
        in_specs=[
            pl.BlockSpec((NP // 2, D), lambda i, j: (i, 0)),
            pl.BlockSpec((1, D, DH), lambda i, j: (j, 0, 0)),
            pl.BlockSpec((1, 1, DH), lambda i, j: (j, 0, 0)),
        ],
        out_specs=pl.BlockSpec((1, NP // 2, DH), lambda i, j: (j, i, 0)),
        out_shape=jax.ShapeDtypeStruct((2, NP, DH), jnp.float32),
    )(xp, Wt_s, b_s)


def _merge_halves(hs):
    def body(p_ref, o_ref):
        o_ref[...] = jnp.concatenate([p_ref[0], p_ref[1]], axis=1)

    return pl.pallas_call(
        body,
        grid=(10,),
        in_specs=[pl.BlockSpec((2, N // 10, DH), lambda i: (0, i, 0))],
        out_specs=pl.BlockSpec((N // 10, D), lambda i: (i, 0)),
        out_shape=jax.ShapeDtypeStruct((N, D), jnp.float32),
    )(hs)


def _propagate(hs, src, dst, w):
    mesh = plsc.VectorSubcoreMesh(core_axis_name="c", subcore_axis_name="s")

    @functools.partial(
        pl.kernel,
        out_type=jax.ShapeDtypeStruct((NC, NP, DH), jnp.float32),
        mesh=mesh,
        compiler_params=pltpu.CompilerParams(use_tc_tiling_on_sc=False),
        scratch_types=[
            pltpu.VMEM((2, CB, B), jnp.int32),
            pltpu.VMEM((2, CB, B), jnp.int32),
            pltpu.VMEM((2, CB * B), jnp.float32),
            pltpu.VMEM((B, DH), jnp.float32),
            pltpu.VMEM((B, DH), jnp.float32),
            pltpu.VMEM_SHARED((NP, DH), jnp.float32),
            pltpu.VMEM_SHARED((NP, DH), jnp.float32),
            pltpu.SemaphoreType.DMA,
            pltpu.SemaphoreType.DMA,
            pltpu.SemaphoreType.DMA,
            pltpu.SemaphoreType.DMA,
            pltpu.SemaphoreType.DMA,
        ],
    )
    def k(hs_hbm, src_hbm, dst_hbm, w_hbm, out_hbm,
          src_c, dst_c, w_c, rows0, rows1, h_sh, acc_sh,
          gs0, gs1, ss0, ss1, cs):
        c = lax.axis_index("c")
        s = lax.axis_index("s")

        row0 = s * ROWS_PER_SUB
        pltpu.sync_copy(hs_hbm.at[c, pl.ds(row0, ROWS_PER_SUB)],
                        h_sh.at[pl.ds(row0, ROWS_PER_SUB)])

        def zrow(i, carry):
            for g in range(DH // 16):
                rows0[i, pl.ds(g * 16, 16)] = jnp.zeros((16,), jnp.float32)
            return carry

        lax.fori_loop(0, B, zrow, 0, unroll=False)
        for t in range(ROWS_PER_SUB // B):
            pltpu.sync_copy(rows0, acc_sh.at[pl.ds(row0 + t * B, B)])
        plsc.subcore_barrier()

        pltpu.sync_copy(src_hbm.at[s, 0], src_c.at[0])
        pltpu.sync_copy(dst_hbm.at[s, 0], dst_c.at[0])
        pltpu.sync_copy(w_hbm.at[s, 0], w_c.at[0])

        def cix(jg):
            return lax.shift_right_logical(jg, 4) & 1, jg & (CB - 1)

        def scale(rows, jg):
            pn, bn = cix(jg)
            base = bn * B

            def grp(eg, carry):
                w16 = w_c[pn, pl.ds(base + eg * 16, 16)]
                for l in range(16):
                    e = eg * 16 + l
                    wv = jnp.full((16,), w16[l])
                    for g in range(DH // 16):
                        rows[e, pl.ds(g * 16, 16)] = (
                            rows[e, pl.ds(g * 16, 16)] * wv)
                return carry

            lax.fori_loop(0, B // 16, grp, 0, unroll=4)

        def gather(rows, sem, jg):
            pn, bn = cix(jg)
            pltpu.async_copy(h_sh.at[src_c.at[pn, bn]], rows, sem)

        def wait_gather(rows, sem, jg):
            pn, bn = cix(jg)
            pltpu.make_async_copy(h_sh.at[src_c.at[pn, bn]], rows, sem).wait()

        def scatter(rows, sem, jg):
            pn, bn = cix(jg)
            pltpu.async_copy(rows, acc_sh.at[dst_c.at[pn, bn]], sem, add=True)

        def wait_scatter(rows, sem, jg):
            pn, bn = cix(jg)
            pltpu.make_async_copy(rows, acc_sh.at[dst_c.at[pn, bn]],
                                  sem).wait()

        gather(rows0, gs0, 0)

        def pair(idx, carry):
            kc, j2 = idx // (CB // 2), idx % (CB // 2)
            jg0 = kc * CB + 2 * j2
            jg1 = jg0 + 1

            wait_gather(rows0, gs0, jg0)

            @pl.when(jg0 > 0)
            def _():
                wait_scatter(rows1, ss1, jg1 - 2)

            @pl.when(jnp.logical_and(j2 == 0, kc < NCHK - 1))
            def _():
                pltpu.async_copy(src_hbm.at[s, kc + 1],
                                 src_c.at[(kc + 1) & 1], cs)
                pltpu.async_copy(dst_hbm.at[s, kc + 1],
                                 dst_c.at[(kc + 1) & 1], cs)
                pltpu.async_copy(w_hbm.at[s, kc + 1],
                                 w_c.at[(kc + 1) & 1], cs)

            gather(rows1, gs1, jg1)
            scale(rows0, jg0)
            scatter(rows0, ss0, jg0)

            wait_gather(rows1, gs1, jg1)

            @pl.when(jg1 < NBLK - 1)
            def _():
                wait_scatter(rows0, ss0, jg0)

                @pl.when(j2 == CB // 2 - 1)
                def _():
                    pltpu.make_async_copy(src_hbm.at[s, kc + 1],
                                          src_c.at[(kc + 1) & 1], cs).wait()
                    pltpu.make_async_copy(dst_hbm.at[s, kc + 1],
                                          dst_c.at[(kc + 1) & 1], cs).wait()
                    pltpu.make_async_copy(w_hbm.at[s, kc + 1],
                                          w_c.at[(kc + 1) & 1], cs).wait()

                gather(rows0, gs0, jg0 + 2)

            scale(rows1, jg1)
            scatter(rows1, ss1, jg1)
            return carry

        lax.fori_loop(0, NBLK // 2, pair, 0, unroll=False)
        wait_scatter(rows0, ss0, NBLK - 2)
        wait_scatter(rows1, ss1, NBLK - 1)

        plsc.subcore_barrier()
        pltpu.sync_copy(acc_sh.at[pl.ds(row0, ROWS_PER_SUB)],
                        out_hbm.at[c, pl.ds(row0, ROWS_PER_SUB)])

    return k(hs, src, dst, w)


def kernel(x, edge_index, edge_weight, W, b):
    src = edge_index[0].astype(jnp.int32)
    dst = edge_index[1].astype(jnp.int32)
    w = edge_weight.astype(jnp.float32)

    pad = EPAD - E
    src = jnp.pad(src, (0, pad)).reshape(NS, NCHK, CB, B)
    dst = jnp.pad(dst, (0, pad)).reshape(NS, NCHK, CB, B)
    w = jnp.pad(w, (0, pad)).reshape(NS, NCHK, CB * B)

    xp = jnp.pad(x, ((0, NP - N), (0, 0)))
    Wt = W.T
    Wt_s = jnp.stack([Wt[:, :DH], Wt[:, DH:]])
    b_s = jnp.stack([b[:DH], b[DH:]]).reshape(2, 1, DH)

    hs = _linear_relu_split(xp, Wt_s, b_s)
    for _ in range(K):
        hs = _propagate(hs, src, dst, w)
    return _merge_halves(hs)


# speedup vs baseline: 1.1454x; 1.1454x over previous
import functools

import jax
import jax.numpy as jnp
from jax import lax
from jax.experimental import pallas as pl
from jax.experimental.pallas import tpu as pltpu
from jax.experimental.pallas import tpu_sc as plsc

N = 10000
D = 128
DH = D // 2
E = 320000
K = 2
NC, NS = 2, 16
B = 128
NBLK = 160
CB = 16
NCHK = NBLK // CB
EPAD = NS * NBLK * B
NP = 10240
ROWS_PER_SUB = NP // NS


def _linear_relu_split(xp, Wt_s, b_s):
    def body(x_ref, w_ref, b_ref, o_ref):
        acc = jnp.dot(x_ref[...], w_ref[0], preferred_element_type=jnp.float32)
        o_ref[0] = jnp.maximum(acc + b_ref[0], 0.0)

    return pl.pallas_call(
        body,
        grid=(2, 2),
        in_specs=[
            pl.BlockSpec((NP // 2, D), lambda i, j: (i, 0)),
            pl.BlockSpec((1, D, DH), lambda i, j: (j, 0, 0)),
            pl.BlockSpec((1, 1, DH), lambda i, j: (j, 0, 0)),
        ],
        out_specs=pl.BlockSpec((1, NP // 2, DH), lambda i, j: (j, i, 0)),
        out_shape=jax.ShapeDtypeStruct((2, NP, DH), jnp.float32),
    )(xp, Wt_s, b_s)


def _merge_halves(hs):
    def body(p_ref, o_ref):
        o_ref[...] = jnp.concatenate([p_ref[0], p_ref[1]], axis=1)

    return pl.pallas_call(
        body,
        grid=(10,),
        in_specs=[pl.BlockSpec((2, N // 10, DH), lambda i: (0, i, 0))],
        out_specs=pl.BlockSpec((N // 10, D), lambda i: (i, 0)),
        out_shape=jax.ShapeDtypeStruct((N, D), jnp.float32),
    )(hs)


def _propagate(hs, src, dst, w):
    mesh = plsc.VectorSubcoreMesh(core_axis_name="c", subcore_axis_name="s")

    @functools.partial(
        pl.kernel,
        out_type=jax.ShapeDtypeStruct((NC, NP, DH), jnp.float32),
        mesh=mesh,
        compiler_params=pltpu.CompilerParams(use_tc_tiling_on_sc=False),
        scratch_types=[
            pltpu.VMEM((2, CB, B), jnp.int32),
            pltpu.VMEM((2, CB, B), jnp.int32),
            pltpu.VMEM((2, CB * B), jnp.float32),
            pltpu.VMEM((B, DH), jnp.float32),
            pltpu.VMEM((B, DH), jnp.float32),
            pltpu.VMEM((B, DH), jnp.float32),
            pltpu.VMEM((B, DH), jnp.float32),
            pltpu.VMEM_SHARED((NP, DH), jnp.float32),
            pltpu.VMEM_SHARED((NP, DH), jnp.float32),
            pltpu.SemaphoreType.DMA,
            pltpu.SemaphoreType.DMA,
            pltpu.SemaphoreType.DMA,
            pltpu.SemaphoreType.DMA,
            pltpu.SemaphoreType.DMA,
            pltpu.SemaphoreType.DMA,
            pltpu.SemaphoreType.DMA,
            pltpu.SemaphoreType.DMA,
            pltpu.SemaphoreType.DMA,
        ],
    )
    def k(hs_hbm, src_hbm, dst_hbm, w_hbm, out_hbm,
          src_c, dst_c, w_c, rows0, rows1, rows2, rows3, h_sh, acc_sh,
          gs0, gs1, gs2, gs3, ss0, ss1, ss2, ss3, cs):
        c = lax.axis_index("c")
        s = lax.axis_index("s")

        row0 = s * ROWS_PER_SUB
        pltpu.sync_copy(hs_hbm.at[c, pl.ds(row0, ROWS_PER_SUB)],
                        h_sh.at[pl.ds(row0, ROWS_PER_SUB)])

        def zrow(i, carry):
            for g in range(DH // 16):
                rows0[i, pl.ds(g * 16, 16)] = jnp.zeros((16,), jnp.float32)
            return carry

        lax.fori_loop(0, B, zrow, 0, unroll=False)
        for t in range(ROWS_PER_SUB // B):
            pltpu.sync_copy(rows0, acc_sh.at[pl.ds(row0 + t * B, B)])
        plsc.subcore_barrier()

        pltpu.sync_copy(src_hbm.at[s, 0], src_c.at[0])
        pltpu.sync_copy(dst_hbm.at[s, 0], dst_c.at[0])
        pltpu.sync_copy(w_hbm.at[s, 0], w_c.at[0])

        def cix(jg):
            return lax.shift_right_logical(jg, 4) & 1, jg & (CB - 1)

        def scale(rows, jg):
            pn, bn = cix(jg)
            base = bn * B

            def grp(eg, carry):
                w16 = w_c[pn, pl.ds(base + eg * 16, 16)]
                for l in range(16):
                    e = eg * 16 + l
                    wv = jnp.full((16,), w16[l])
                    for g in range(DH // 16):
                        rows[e, pl.ds(g * 16, 16)] = (
                            rows[e, pl.ds(g * 16, 16)] * wv)
                return carry

            lax.fori_loop(0, B // 16, grp, 0, unroll=4)

        def gather(rows, sem, jg):
            pn, bn = cix(jg)
            pltpu.async_copy(h_sh.at[src_c.at[pn, bn]], rows, sem)

        def wait_gather(rows, sem, jg):
            pn, bn = cix(jg)
            pltpu.make_async_copy(h_sh.at[src_c.at[pn, bn]], rows, sem).wait()

        def scatter(rows, sem, jg):
            pn, bn = cix(jg)
            pltpu.async_copy(rows, acc_sh.at[dst_c.at[pn, bn]], sem, add=True)

        def wait_scatter(rows, sem, jg):
            pn, bn = cix(jg)
            pltpu.make_async_copy(rows, acc_sh.at[dst_c.at[pn, bn]],
                                  sem).wait()

        rows = [rows0, rows1, rows2, rows3]
        gs = [gs0, gs1, gs2, gs3]
        ss = [ss0, ss1, ss2, ss3]

        gather(rows0, gs0, 0)
        gather(rows1, gs1, 1)

        def quad(q, carry):
            kc = lax.shift_right_logical(q, 2)
            qm = q & 3
            for m in range(4):
                jg = 4 * q + m
                wait_gather(rows[m], gs[m], jg)

                mp = (m + 2) % 4
                if m >= 2:
                    wait_scatter(rows[mp], ss[mp], jg - 2)
                else:
                    @pl.when(jg >= 2)
                    def _(mp=mp, jg=jg):
                        wait_scatter(rows[mp], ss[mp], jg - 2)

                if m == 2:
                    @pl.when(jnp.logical_and(qm == 0, kc < NCHK - 1))
                    def _():
                        pltpu.async_copy(src_hbm.at[s, kc + 1],
                                         src_c.at[(kc + 1) & 1], cs)
                        pltpu.async_copy(dst_hbm.at[s, kc + 1],
                                         dst_c.at[(kc + 1) & 1], cs)
                        pltpu.async_copy(w_hbm.at[s, kc + 1],
                                         w_c.at[(kc + 1) & 1], cs)

                    @pl.when(jnp.logical_and(qm == 3, kc < NCHK - 1))
                    def _():
                        pltpu.make_async_copy(src_hbm.at[s, kc + 1],
                                              src_c.at[(kc + 1) & 1],
                                              cs).wait()
                        pltpu.make_async_copy(dst_hbm.at[s, kc + 1],
                                              dst_c.at[(kc + 1) & 1],
                                              cs).wait()
                        pltpu.make_async_copy(w_hbm.at[s, kc + 1],
                                              w_c.at[(kc + 1) & 1],
                                              cs).wait()

                @pl.when(jg + 2 < NBLK)
                def _(mp=mp, jg=jg):
                    gather(rows[mp], gs[mp], jg + 2)

                scale(rows[m], jg)
                scatter(rows[m], ss[m], jg)
            return carry

        lax.fori_loop(0, NBLK // 4, quad, 0, unroll=False)
        wait_scatter(rows2, ss2, NBLK - 2)
        wait_scatter(rows3, ss3, NBLK - 1)

        plsc.subcore_barrier()
        pltpu.sync_copy(acc_sh.at[pl.ds(row0, ROWS_PER_SUB)],
                        out_hbm.at[c, pl.ds(row0, ROWS_PER_SUB)])

    return k(hs, src, dst, w)


def kernel(x, edge_index, edge_weight, W, b):
    src = edge_index[0].astype(jnp.int32)
    dst = edge_index[1].astype(jnp.int32)
    w = edge_weight.astype(jnp.float32)

    pad = EPAD - E
    src = jnp.pad(src, (0, pad)).reshape(NS, NCHK, CB, B)
    dst = jnp.pad(dst, (0, pad)).reshape(NS, NCHK, CB, B)
    w = jnp.pad(w, (0, pad)).reshape(NS, NCHK, CB * B)

    xp = jnp.pad(x, ((0, NP - N), (0, 0)))
    Wt = W.T
    Wt_s = jnp.stack([Wt[:, :DH], Wt[:, DH:]])
    b_s = jnp.stack([b[:DH], b[DH:]]).reshape(2, 1, DH)

    hs = _linear_relu_split(xp, Wt_s, b_s)
    for _ in range(K):
        hs = _propagate(hs, src, dst, w)
    return _merge_halves(hs)
